# baseline (speedup 1.0000x reference)
import functools

import jax
import jax.numpy as jnp
from jax import lax
from jax.experimental import pallas as pl
from jax.experimental.pallas import tpu as pltpu
from jax.experimental.pallas import tpu_sc as plsc

NUM_CORES = 2
NUM_SUBCORES = 16
NUM_WORKERS = NUM_CORES * NUM_SUBCORES
LANES = 16

CHUNK = 8
NBUF = 4


def kernel(pos_idxs, pos_emb):
    B, S = pos_idxs.shape
    V, D = pos_emb.shape
    n_idx = B * S
    per_worker = n_idx // NUM_WORKERS
    n_chunks = per_worker // CHUNK

    idx_flat = pos_idxs.reshape(n_idx).astype(jnp.int32)

    mesh = plsc.VectorSubcoreMesh(core_axis_name="c", subcore_axis_name="s")

    @functools.partial(
        pl.kernel,
        mesh=mesh,
        out_type=jax.ShapeDtypeStruct((n_idx, D), jnp.float32),
        scratch_types=(
            [pltpu.VMEM((per_worker,), jnp.int32)]
            + [pltpu.VMEM_SHARED((NUM_SUBCORES, NBUF, CHUNK, D), jnp.float32)]
            + [pltpu.VMEM((CHUNK, D), jnp.float32) for _ in range(NBUF)]
            + [pltpu.SemaphoreType.DMA for _ in range(3 * NBUF)]
        ),
    )
    def gather_kernel(table_hbm, idx_hbm, out_hbm, idx_v, spmem, *rest):
        bufs = rest[:NBUF]
        sg = rest[NBUF : 2 * NBUF]
        ss = rest[2 * NBUF : 3 * NBUF]
        swr = rest[3 * NBUF :]

        sid = lax.axis_index("s")
        wid = sid * NUM_CORES + lax.axis_index("c")
        base = wid * per_worker
        pltpu.sync_copy(idx_hbm.at[pl.ds(base, per_worker)], idx_v)

        @pl.loop(0, per_worker, step=LANES)
        def _(o):
            v = idx_v[pl.ds(o, LANES)]
            idx_v[pl.ds(o, LANES)] = jnp.minimum(jnp.maximum(v, 0), V - 1)

        def start_g(c, k):
            pltpu.async_copy(
                table_hbm.at[idx_v.at[pl.ds(c * CHUNK, CHUNK)]], bufs[k], sg[k]
            )

        def wait_g(k):
            pltpu.make_async_copy(out_hbm.at[pl.ds(base, CHUNK)], bufs[k], sg[k]).wait()

        def start_s(k):
            pltpu.async_copy(bufs[k], spmem.at[sid, k], ss[k])

        def wait_s(k):
            pltpu.make_async_copy(bufs[k], spmem.at[sid, k], ss[k]).wait()

        def start_w(c, k):
            pltpu.async_copy(
                spmem.at[sid, k], out_hbm.at[pl.ds(base + c * CHUNK, CHUNK)], swr[k]
            )

        def wait_w(k):
            pltpu.make_async_copy(
                spmem.at[sid, k], out_hbm.at[pl.ds(base, CHUNK)], swr[k]
            ).wait()

        for k in range(NBUF):
            start_g(k, k)
        for k in range(NBUF):
            wait_g(k)
            start_s(k)
        for k in range(NBUF):
            wait_s(k)
            start_w(k, k)
            start_g(NBUF + k, k)

        @pl.loop(NBUF, n_chunks - NBUF, step=NBUF)
        def _(c):
            for k in range(NBUF):
                wait_w(k)
                wait_g(k)
                start_s(k)
            for k in range(NBUF):
                wait_s(k)
                start_w(c + k, k)
                start_g(c + k + NBUF, k)

        for k in range(NBUF):
            wait_w(k)
            wait_g(k)
            start_s(k)
        for k in range(NBUF):
            wait_s(k)
            start_w(n_chunks - NBUF + k, k)
        for k in range(NBUF):
            wait_w(k)

    out = gather_kernel(pos_emb, idx_flat)
    return out.reshape(B, S, D)


# speedup vs baseline: 1.0045x; 1.0045x over previous
import functools

import jax
import jax.numpy as jnp
from jax import lax
from jax.experimental import pallas as pl
from jax.experimental.pallas import tpu as pltpu
from jax.experimental.pallas import tpu_sc as plsc

NUM_CORES = 2
NUM_SUBCORES = 16
NUM_WORKERS = NUM_CORES * NUM_SUBCORES
LANES = 16

CHUNK = 8
NGBUF = 8
NBUF = 4


def kernel(pos_idxs, pos_emb):
    B, S = pos_idxs.shape
    V, D = pos_emb.shape
    n_idx = B * S
    per_worker = n_idx // NUM_WORKERS
    n_chunks = per_worker // CHUNK

    idx_flat = pos_idxs.reshape(n_idx).astype(jnp.int32)

    mesh = plsc.VectorSubcoreMesh(core_axis_name="c", subcore_axis_name="s")

    @functools.partial(
        pl.kernel,
        mesh=mesh,
        out_type=jax.ShapeDtypeStruct((n_idx, D), jnp.float32),
        scratch_types=(
            [pltpu.VMEM((per_worker,), jnp.int32)]
            + [pltpu.VMEM_SHARED((NUM_SUBCORES, NBUF, CHUNK, D), jnp.float32)]
            + [pltpu.VMEM((CHUNK, D), jnp.float32) for _ in range(NGBUF)]
            + [pltpu.SemaphoreType.DMA for _ in range(2 * NGBUF + NBUF)]
        ),
    )
    def gather_kernel(table_hbm, idx_hbm, out_hbm, idx_v, spmem, *rest):
        bufs = rest[:NGBUF]
        sg = rest[NGBUF : 2 * NGBUF]
        ss = rest[2 * NGBUF : 3 * NGBUF]
        swr = rest[3 * NGBUF :]

        sid = lax.axis_index("s")
        wid = sid * NUM_CORES + lax.axis_index("c")
        base = wid * per_worker
        pltpu.sync_copy(idx_hbm.at[pl.ds(base, per_worker)], idx_v)

        @pl.loop(0, per_worker, step=LANES)
        def _(o):
            v = idx_v[pl.ds(o, LANES)]
            idx_v[pl.ds(o, LANES)] = jnp.minimum(jnp.maximum(v, 0), V - 1)

        def start_g(c, k):
            pltpu.async_copy(
                table_hbm.at[idx_v.at[pl.ds(c * CHUNK, CHUNK)]], bufs[k], sg[k]
            )

        def wait_g(k):
            pltpu.make_async_copy(out_hbm.at[pl.ds(base, CHUNK)], bufs[k], sg[k]).wait()

        def start_s(b, t):
            pltpu.async_copy(bufs[b], spmem.at[sid, t], ss[b])

        def wait_s(b, t):
            pltpu.make_async_copy(bufs[b], spmem.at[sid, t], ss[b]).wait()

        def start_w(c, t):
            pltpu.async_copy(
                spmem.at[sid, t], out_hbm.at[pl.ds(base + c * CHUNK, CHUNK)], swr[t]
            )

        def wait_w(t):
            pltpu.make_async_copy(
                spmem.at[sid, t], out_hbm.at[pl.ds(base, CHUNK)], swr[t]
            ).wait()

        for k in range(NGBUF):
            start_g(k, k)
        for j in range(NGBUF):
            t = j % NBUF
            wait_g(j)
            if j >= NBUF:
                wait_w(t)
            start_s(j, t)
            wait_s(j, t)
            start_w(j, t)
            start_g(j + NGBUF, j)

        @pl.loop(NGBUF, n_chunks - NGBUF, step=NGBUF)
        def _(c):
            for k in range(NGBUF):
                t = k % NBUF
                wait_g(k)
                wait_w(t)
                start_s(k, t)
                wait_s(k, t)
                start_w(c + k, t)
                start_g(c + k + NGBUF, k)

        for k in range(NGBUF):
            t = k % NBUF
            wait_g(k)
            wait_w(t)
            start_s(k, t)
            wait_s(k, t)
            start_w(n_chunks - NGBUF + k, t)
        for t in range(NBUF):
            wait_w(t)

    out = gather_kernel(pos_emb, idx_flat)
    return out.reshape(B, S, D)
